# baseline (speedup 1.0000x reference)
import functools

import jax
import jax.numpy as jnp
from jax import lax
from jax.experimental import pallas as pl
from jax.experimental.pallas import tpu as pltpu
from jax.experimental.pallas import tpu_sc as plsc

NUM_EMB = 100000
D = 1024
LANES = 16
VECS_PER_ROW = D // LANES

NUM_CORES = 2
NUM_SUBCORES = 16
NW = NUM_CORES * NUM_SUBCORES

B_TOTAL = 4 * 4096
B_PER_W = B_TOTAL // NW
CHUNK = 32
N_CHUNKS = B_PER_W // CHUNK

SCALE = 32.0

_MESH = plsc.VectorSubcoreMesh(
    core_axis_name="c", subcore_axis_name="s",
    num_cores=NUM_CORES, num_subcores=NUM_SUBCORES,
)


@functools.partial(
    pl.kernel,
    out_type=jax.ShapeDtypeStruct((B_TOTAL, D), jnp.float32),
    mesh=_MESH,
    scratch_types=[
        pltpu.VMEM((B_PER_W,), jnp.int32),
        pltpu.VMEM((CHUNK, D), jnp.float32),
        pltpu.VMEM((CHUNK, D), jnp.float32),
        pltpu.VMEM((CHUNK, D), jnp.float32),
        pltpu.SemaphoreType.DMA,
        pltpu.SemaphoreType.DMA,
        pltpu.SemaphoreType.DMA,
        pltpu.SemaphoreType.DMA,
        pltpu.SemaphoreType.DMA,
        pltpu.SemaphoreType.DMA,
    ],
)
def _gather_scale(ids_hbm, w_hbm, out_hbm, idx_v,
                  buf0, buf1, buf2, g0, g1, g2, s0, s1, s2):
    wid = lax.axis_index("s") * NUM_CORES + lax.axis_index("c")
    base = wid * B_PER_W
    row = wid // (4096 // B_PER_W)
    col = (wid % (4096 // B_PER_W)) * B_PER_W
    pltpu.sync_copy(ids_hbm.at[row, pl.ds(col, B_PER_W)], idx_v)

    bufs = (buf0, buf1, buf2)
    gsems = (g0, g1, g2)
    ssems = (s0, s1, s2)
    NBUF = 3

    HALF = CHUNK // 2

    def scale_half(buf, h):
        def row_body(r, _):
            def col_body(v, _):
                sl = (r, pl.ds(v * LANES, LANES))
                buf[sl] = buf[sl] * SCALE
                return 0

            return lax.fori_loop(0, VECS_PER_ROW, col_body, 0, unroll=8)

        lax.fori_loop(h * HALF, (h + 1) * HALF, row_body, 0)

    def gather(ci):
        b = ci % NBUF
        return pltpu.async_copy(
            w_hbm.at[idx_v.at[pl.ds(ci * CHUNK, CHUNK)]], bufs[b], gsems[b]
        )

    def store_half(ci, h):
        b = ci % NBUF
        return pltpu.async_copy(
            bufs[b].at[pl.ds(h * HALF, HALF)],
            out_hbm.at[pl.ds(base + ci * CHUNK + h * HALF, HALF)],
            ssems[b],
        )

    gd = [None] * N_CHUNKS
    sd = [None] * N_CHUNKS
    gd[0] = gather(0)
    gd[1] = gather(1)
    for ci in range(N_CHUNKS):
        b = ci % NBUF
        if ci >= 1:
            sd[ci - 1][0].wait()
            sd[ci - 1][1].wait()
        if ci + 2 < N_CHUNKS:
            gd[ci + 2] = gather(ci + 2)
        gd[ci].wait()
        scale_half(bufs[b], 0)
        s_lo = store_half(ci, 0)
        scale_half(bufs[b], 1)
        s_hi = store_half(ci, 1)
        sd[ci] = (s_lo, s_hi)
    sd[N_CHUNKS - 1][0].wait()
    sd[N_CHUNKS - 1][1].wait()


def kernel(input_ids, weight):
    out = _gather_scale(input_ids, weight)
    return out.reshape(input_ids.shape[0], input_ids.shape[1], D)


# speedup vs baseline: 1.0111x; 1.0111x over previous
import functools

import jax
import jax.numpy as jnp
from jax import lax
from jax.experimental import pallas as pl
from jax.experimental.pallas import tpu as pltpu
from jax.experimental.pallas import tpu_sc as plsc

NUM_EMB = 100000
D = 1024
LANES = 16
VECS_PER_ROW = D // LANES

NUM_CORES = 2
NUM_SUBCORES = 16
NW = NUM_CORES * NUM_SUBCORES

B_TOTAL = 4 * 4096
B_PER_W = B_TOTAL // NW
CHUNK = 32
N_CHUNKS = B_PER_W // CHUNK

SCALE = 32.0

_MESH = plsc.VectorSubcoreMesh(
    core_axis_name="c", subcore_axis_name="s",
    num_cores=NUM_CORES, num_subcores=NUM_SUBCORES,
)


@functools.partial(
    pl.kernel,
    out_type=jax.ShapeDtypeStruct((B_TOTAL, D), jnp.float32),
    mesh=_MESH,
    scratch_types=[
        pltpu.VMEM((B_PER_W,), jnp.int32),
        pltpu.VMEM((CHUNK, D), jnp.float32),
        pltpu.VMEM((CHUNK, D), jnp.float32),
        pltpu.VMEM((CHUNK, D), jnp.float32),
        pltpu.SemaphoreType.DMA,
        pltpu.SemaphoreType.DMA,
        pltpu.SemaphoreType.DMA,
        pltpu.SemaphoreType.DMA,
        pltpu.SemaphoreType.DMA,
        pltpu.SemaphoreType.DMA,
    ],
)
def _gather_scale(ids_hbm, w_hbm, out_hbm, idx_v,
                  buf0, buf1, buf2, g0, g1, g2, s0, s1, s2):
    wid = lax.axis_index("s") * NUM_CORES + lax.axis_index("c")
    base = wid * B_PER_W
    row = wid // (4096 // B_PER_W)
    col = (wid % (4096 // B_PER_W)) * B_PER_W
    pltpu.sync_copy(ids_hbm.at[row, pl.ds(col, B_PER_W)], idx_v)

    bufs = (buf0, buf1, buf2)
    gsems = (g0, g1, g2)
    ssems = (s0, s1, s2)
    NBUF = 3

    HALF = CHUNK // 2

    def scale_half(buf, h):
        lo = h * HALF * VECS_PER_ROW
        hi = (h + 1) * HALF * VECS_PER_ROW

        @plsc.parallel_loop(lo, hi, step=1, unroll=8)
        def _(i):
            r = lax.shift_right_logical(i, 6)
            c = lax.mul(lax.bitwise_and(i, VECS_PER_ROW - 1), LANES)
            sl = (r, pl.ds(c, LANES))
            buf[sl] = buf[sl] * SCALE

    def gather(ci):
        b = ci % NBUF
        return pltpu.async_copy(
            w_hbm.at[idx_v.at[pl.ds(ci * CHUNK, CHUNK)]], bufs[b], gsems[b]
        )

    def store_half(ci, h):
        b = ci % NBUF
        return pltpu.async_copy(
            bufs[b].at[pl.ds(h * HALF, HALF)],
            out_hbm.at[pl.ds(base + ci * CHUNK + h * HALF, HALF)],
            ssems[b],
        )

    gd = [None] * N_CHUNKS
    sd = [None] * N_CHUNKS
    gd[0] = gather(0)
    gd[1] = gather(1)
    for ci in range(N_CHUNKS):
        b = ci % NBUF
        if ci >= 1:
            sd[ci - 1][0].wait()
            sd[ci - 1][1].wait()
        if ci + 2 < N_CHUNKS:
            gd[ci + 2] = gather(ci + 2)
        gd[ci].wait()
        scale_half(bufs[b], 0)
        s_lo = store_half(ci, 0)
        scale_half(bufs[b], 1)
        s_hi = store_half(ci, 1)
        sd[ci] = (s_lo, s_hi)
    sd[N_CHUNKS - 1][0].wait()
    sd[N_CHUNKS - 1][1].wait()


def kernel(input_ids, weight):
    out = _gather_scale(input_ids, weight)
    return out.reshape(input_ids.shape[0], input_ids.shape[1], D)


# speedup vs baseline: 1.0301x; 1.0188x over previous
import functools

import jax
import jax.numpy as jnp
from jax import lax
from jax.experimental import pallas as pl
from jax.experimental.pallas import tpu as pltpu
from jax.experimental.pallas import tpu_sc as plsc

NUM_EMB = 100000
D = 1024
LANES = 16
VECS_PER_ROW = D // LANES

NUM_CORES = 2
NUM_SUBCORES = 16
NW = NUM_CORES * NUM_SUBCORES

B_TOTAL = 4 * 4096
B_PER_W = B_TOTAL // NW
CHUNK = 16
N_CHUNKS = B_PER_W // CHUNK
NBUF = 6
LOOKAHEAD = 4

SCALE = 32.0

_MESH = plsc.VectorSubcoreMesh(
    core_axis_name="c", subcore_axis_name="s",
    num_cores=NUM_CORES, num_subcores=NUM_SUBCORES,
)


@functools.partial(
    pl.kernel,
    out_type=jax.ShapeDtypeStruct((B_TOTAL, D), jnp.float32),
    mesh=_MESH,
    scratch_types=[
        pltpu.VMEM((B_PER_W,), jnp.int32),
    ]
    + [pltpu.VMEM((CHUNK, D), jnp.float32)] * NBUF
    + [pltpu.SemaphoreType.DMA] * (2 * NBUF),
)
def _gather_scale(ids_hbm, w_hbm, out_hbm, idx_v, *bufs_and_sems):
    wid = lax.axis_index("s") * NUM_CORES + lax.axis_index("c")
    base = wid * B_PER_W
    row = wid // (4096 // B_PER_W)
    col = (wid % (4096 // B_PER_W)) * B_PER_W
    pltpu.sync_copy(ids_hbm.at[row, pl.ds(col, B_PER_W)], idx_v)

    bufs = bufs_and_sems[:NBUF]
    gsems = bufs_and_sems[NBUF:2 * NBUF]
    ssems = bufs_and_sems[2 * NBUF:]

    def scale_chunk(buf):
        @plsc.parallel_loop(0, CHUNK * VECS_PER_ROW, step=1, unroll=8)
        def _(i):
            r = lax.shift_right_logical(i, 6)
            c = lax.mul(lax.bitwise_and(i, VECS_PER_ROW - 1), LANES)
            sl = (r, pl.ds(c, LANES))
            buf[sl] = buf[sl] * SCALE

    def gather(ci):
        b = ci % NBUF
        return pltpu.async_copy(
            w_hbm.at[idx_v.at[pl.ds(ci * CHUNK, CHUNK)]], bufs[b], gsems[b]
        )

    def store(ci):
        b = ci % NBUF
        return pltpu.async_copy(
            bufs[b], out_hbm.at[pl.ds(base + ci * CHUNK, CHUNK)], ssems[b]
        )

    gd = [None] * N_CHUNKS
    sd = [None] * N_CHUNKS
    for ci in range(LOOKAHEAD):
        gd[ci] = gather(ci)
    for ci in range(N_CHUNKS):
        b = ci % NBUF
        nxt = ci + LOOKAHEAD
        if nxt < N_CHUNKS:
            if nxt >= NBUF:
                sd[nxt - NBUF].wait()
            gd[nxt] = gather(nxt)
        gd[ci].wait()
        scale_chunk(bufs[b])
        sd[ci] = store(ci)
    for ci in range(N_CHUNKS - NBUF, N_CHUNKS):
        sd[ci].wait()


def kernel(input_ids, weight):
    out = _gather_scale(input_ids, weight)
    return out.reshape(input_ids.shape[0], input_ids.shape[1], D)


# speedup vs baseline: 1.0450x; 1.0145x over previous
import functools

import jax
import jax.numpy as jnp
from jax import lax
from jax.experimental import pallas as pl
from jax.experimental.pallas import tpu as pltpu
from jax.experimental.pallas import tpu_sc as plsc

NUM_EMB = 100000
D = 1024
LANES = 16
VECS_PER_ROW = D // LANES

NUM_CORES = 2
NUM_SUBCORES = 16
NW = NUM_CORES * NUM_SUBCORES

B_TOTAL = 4 * 4096
B_PER_W = B_TOTAL // NW
CHUNK = 16
N_CHUNKS = B_PER_W // CHUNK
NBUF = 7
LOOKAHEAD = 5

SCALE = 32.0

_MESH = plsc.VectorSubcoreMesh(
    core_axis_name="c", subcore_axis_name="s",
    num_cores=NUM_CORES, num_subcores=NUM_SUBCORES,
)


@functools.partial(
    pl.kernel,
    out_type=jax.ShapeDtypeStruct((B_TOTAL, D), jnp.float32),
    mesh=_MESH,
    scratch_types=[
        pltpu.VMEM((B_PER_W,), jnp.int32),
    ]
    + [pltpu.VMEM((CHUNK, D), jnp.float32)] * NBUF
    + [pltpu.SemaphoreType.DMA] * (2 * NBUF),
)
def _gather_scale(ids_hbm, w_hbm, out_hbm, idx_v, *bufs_and_sems):
    wid = lax.axis_index("s") * NUM_CORES + lax.axis_index("c")
    base = wid * B_PER_W
    row = wid // (4096 // B_PER_W)
    col = (wid % (4096 // B_PER_W)) * B_PER_W
    pltpu.sync_copy(ids_hbm.at[row, pl.ds(col, B_PER_W)], idx_v)

    bufs = bufs_and_sems[:NBUF]
    gsems = bufs_and_sems[NBUF:2 * NBUF]
    ssems = bufs_and_sems[2 * NBUF:]

    def scale_chunk(buf):
        @plsc.parallel_loop(0, CHUNK * VECS_PER_ROW, step=1, unroll=8)
        def _(i):
            r = lax.shift_right_logical(i, 6)
            c = lax.mul(lax.bitwise_and(i, VECS_PER_ROW - 1), LANES)
            sl = (r, pl.ds(c, LANES))
            buf[sl] = buf[sl] * SCALE

    def gather(ci):
        b = ci % NBUF
        return pltpu.async_copy(
            w_hbm.at[idx_v.at[pl.ds(ci * CHUNK, CHUNK)]], bufs[b], gsems[b]
        )

    def store(ci):
        b = ci % NBUF
        return pltpu.async_copy(
            bufs[b], out_hbm.at[pl.ds(base + ci * CHUNK, CHUNK)], ssems[b]
        )

    gd = [None] * N_CHUNKS
    sd = [None] * N_CHUNKS
    for ci in range(LOOKAHEAD):
        gd[ci] = gather(ci)
    for ci in range(N_CHUNKS):
        b = ci % NBUF
        nxt = ci + LOOKAHEAD
        if nxt < N_CHUNKS:
            if nxt >= NBUF:
                sd[nxt - NBUF].wait()
            gd[nxt] = gather(nxt)
        gd[ci].wait()
        scale_chunk(bufs[b])
        sd[ci] = store(ci)
    for ci in range(N_CHUNKS - NBUF, N_CHUNKS):
        sd[ci].wait()


def kernel(input_ids, weight):
    out = _gather_scale(input_ids, weight)
    return out.reshape(input_ids.shape[0], input_ids.shape[1], D)
